# baseline (speedup 1.0000x reference)
import functools

import jax
import jax.numpy as jnp
from jax import lax
from jax.experimental import pallas as pl
from jax.experimental.pallas import tpu as pltpu
from jax.experimental.pallas import tpu_sc as plsc

H = 768
NW = 32
CHUNK = 64
TBLK = 2048
STAGE_BATCHES = (2, 2)


def _make_sc_gather(n_tokens, stage_tokens, base0):
    per_w = stage_tokens // NW
    chunk = min(32, per_w // 2)
    n_chunks = per_w // chunk
    assert n_chunks * chunk == per_w and n_chunks >= 2
    mesh = plsc.VectorSubcoreMesh(core_axis_name="c", subcore_axis_name="s")

    @functools.partial(
        pl.kernel,
        out_type=jax.ShapeDtypeStruct((stage_tokens, H), jnp.float32),
        mesh=mesh,
        compiler_params=pltpu.CompilerParams(
            use_tc_tiling_on_sc=True, needs_layout_passes=False),
        scratch_types=[
            pltpu.VMEM((per_w,), jnp.int32),
            pltpu.VMEM((chunk, H), jnp.float32),
            pltpu.VMEM((chunk, H), jnp.float32),
            pltpu.SemaphoreType.DMA,
            pltpu.SemaphoreType.DMA,
            pltpu.SemaphoreType.DMA,
            pltpu.SemaphoreType.DMA,
        ],
    )
    def sc_gather(tok_hbm, wemb_hbm, out_hbm, idx_v, buf0, buf1,
                  si0, si1, so0, so1):
        wid = lax.axis_index("s") * 2 + lax.axis_index("c")
        base = wid * per_w
        seq = tok_hbm.shape[1]
        row = base0 // seq + wid // (seq // per_w)
        col = lax.rem(wid, seq // per_w) * per_w
        pltpu.sync_copy(tok_hbm.at[row, pl.ds(col, per_w)], idx_v)

        bufs = (buf0, buf1)
        sin = (si0, si1)
        sout = (so0, so1)

        def gather_in(c):
            return pltpu.async_copy(
                wemb_hbm.at[idx_v.at[pl.ds(c * chunk, chunk)]],
                bufs[c % 2], sin[c % 2])

        def copy_out(c):
            return pltpu.async_copy(
                bufs[c % 2], out_hbm.at[pl.ds(base + c * chunk, chunk)],
                sout[c % 2])

        ins = [gather_in(0), gather_in(1)]
        outs = [None, None]
        for c in range(n_chunks):
            ins[c % 2].wait()
            outs[c % 2] = copy_out(c)
            if c + 2 < n_chunks:
                outs[c % 2].wait()
                ins[c % 2] = gather_in(c + 2)
        outs[(n_chunks - 2) % 2].wait()
        outs[(n_chunks - 1) % 2].wait()

    return sc_gather


def _ln_body(gref, ttref, pref, tyref, gam, bet, *prev_and_out):
    *prev_ref, oref = prev_and_out
    x = gref[...] + pref[...]
    ttf = ttref[0, 0, :].astype(jnp.float32)
    ty0 = tyref[0, :]
    dty = tyref[1, :] - ty0
    x = x + ty0[None, :] + ttf[:, None] * dty[None, :]
    mean = jnp.mean(x, axis=-1, keepdims=True)
    var = jnp.mean(x * x, axis=-1, keepdims=True) - mean * mean
    inv = lax.rsqrt(var + 1e-12)
    oref[...] = (x - mean) * inv * gam[...] + bet[...]


def _make_tc_ln(n_tokens, stage_tokens, seq, types, blk0, with_prev):
    stage_batch = stage_tokens // seq
    pos_blocks = seq // TBLK
    in_specs = [
        pl.BlockSpec((TBLK, H), lambda p, b: (b * pos_blocks + p, 0)),
        pl.BlockSpec((1, 1, TBLK),
                     lambda p, b: (blk0 + b * pos_blocks + p, 0, 0)),
        pl.BlockSpec((TBLK, H), lambda p, b: (p, 0)),
        pl.BlockSpec((types, H), lambda p, b: (0, 0)),
        pl.BlockSpec((1, H), lambda p, b: (0, 0)),
        pl.BlockSpec((1, H), lambda p, b: (0, 0)),
    ]
    if with_prev:
        in_specs.append(pl.BlockSpec(memory_space=pl.ANY))
    return pl.pallas_call(
        _ln_body,
        grid=(pos_blocks, stage_batch),
        in_specs=in_specs,
        out_specs=pl.BlockSpec(
            (TBLK, H), lambda p, b: (blk0 + b * pos_blocks + p, 0)),
        out_shape=jax.ShapeDtypeStruct((n_tokens, H), jnp.float32),
        input_output_aliases={6: 0} if with_prev else {},
    )


def kernel(token_ids, token_type_ids, word_emb, pos_emb, type_emb, gamma, beta):
    batch, seq = token_ids.shape
    vocab, hidden = word_emb.shape
    types = type_emb.shape[0]
    n_tokens = batch * seq

    tok = token_ids.astype(jnp.int32)
    tt3 = token_type_ids.reshape(n_tokens // TBLK, 1, TBLK).astype(jnp.int32)
    gam = gamma.reshape(1, hidden)
    bet = beta.reshape(1, hidden)

    stage_tok = [b * seq for b in STAGE_BATCHES]
    assert sum(stage_tok) == n_tokens
    bases = [sum(stage_tok[:h]) for h in range(len(stage_tok))]

    gathered = [
        _make_sc_gather(n_tokens, stage_tok[h], bases[h])(tok, word_emb)
        for h in range(len(stage_tok))]

    out = None
    for h in range(len(stage_tok)):
        args = [gathered[h], tt3, pos_emb, type_emb, gam, bet]
        if out is not None:
            args.append(out)
        out = _make_tc_ln(n_tokens, stage_tok[h], seq, types,
                          bases[h] // TBLK, out is not None)(*args)
    return out.reshape(batch, seq, hidden)


# speedup vs baseline: 1.0324x; 1.0324x over previous
import functools

import jax
import jax.numpy as jnp
from jax import lax
from jax.experimental import pallas as pl
from jax.experimental.pallas import tpu as pltpu
from jax.experimental.pallas import tpu_sc as plsc

H = 768
NW = 32
CHUNK = 64
TBLK = 2048
STAGE_BATCHES = (2, 2)


def _make_sc_gather(n_tokens, stage_tokens, base0):
    per_w = stage_tokens // NW
    chunk = min(CHUNK, per_w // 2)
    n_chunks = per_w // chunk
    assert n_chunks * chunk == per_w and n_chunks >= 2
    mesh = plsc.VectorSubcoreMesh(core_axis_name="c", subcore_axis_name="s")

    @functools.partial(
        pl.kernel,
        out_type=jax.ShapeDtypeStruct((stage_tokens, H), jnp.float32),
        mesh=mesh,
        compiler_params=pltpu.CompilerParams(
            use_tc_tiling_on_sc=True, needs_layout_passes=False),
        scratch_types=[
            pltpu.VMEM((per_w,), jnp.int32),
            pltpu.VMEM((chunk, H), jnp.float32),
            pltpu.VMEM((chunk, H), jnp.float32),
            pltpu.SemaphoreType.DMA,
            pltpu.SemaphoreType.DMA,
            pltpu.SemaphoreType.DMA,
            pltpu.SemaphoreType.DMA,
        ],
    )
    def sc_gather(tok_hbm, wemb_hbm, out_hbm, idx_v, buf0, buf1,
                  si0, si1, so0, so1):
        wid = lax.axis_index("s") * 2 + lax.axis_index("c")
        base = wid * per_w
        seq = tok_hbm.shape[1]
        row = base0 // seq + wid // (seq // per_w)
        col = lax.rem(wid, seq // per_w) * per_w
        pltpu.sync_copy(tok_hbm.at[row, pl.ds(col, per_w)], idx_v)

        bufs = (buf0, buf1)
        sin = (si0, si1)
        sout = (so0, so1)

        def gather_in(c):
            return pltpu.async_copy(
                wemb_hbm.at[idx_v.at[pl.ds(c * chunk, chunk)]],
                bufs[c % 2], sin[c % 2])

        def copy_out(c):
            return pltpu.async_copy(
                bufs[c % 2], out_hbm.at[pl.ds(base + c * chunk, chunk)],
                sout[c % 2])

        ins = [gather_in(0), gather_in(1)]
        outs = [None, None]
        for c in range(n_chunks):
            ins[c % 2].wait()
            outs[c % 2] = copy_out(c)
            if c + 2 < n_chunks:
                outs[c % 2].wait()
                ins[c % 2] = gather_in(c + 2)
        outs[(n_chunks - 2) % 2].wait()
        outs[(n_chunks - 1) % 2].wait()

    return sc_gather


def _ln_body(gref, ttref, pref, tyref, gam, bet, *prev_and_out):
    *prev_ref, oref = prev_and_out
    x = gref[...] + pref[...]
    ttf = ttref[0, 0, :].astype(jnp.float32)
    ty0 = tyref[0, :]
    dty = tyref[1, :] - ty0
    x = x + ty0[None, :] + ttf[:, None] * dty[None, :]
    mean = jnp.mean(x, axis=-1, keepdims=True)
    var = jnp.mean(x * x, axis=-1, keepdims=True) - mean * mean
    inv = lax.rsqrt(var + 1e-12)
    oref[...] = (x - mean) * inv * gam[...] + bet[...]


def _make_tc_ln(n_tokens, stage_tokens, seq, types, blk0, with_prev):
    stage_batch = stage_tokens // seq
    pos_blocks = seq // TBLK
    in_specs = [
        pl.BlockSpec((TBLK, H), lambda p, b: (b * pos_blocks + p, 0)),
        pl.BlockSpec((1, 1, TBLK),
                     lambda p, b: (blk0 + b * pos_blocks + p, 0, 0)),
        pl.BlockSpec((TBLK, H), lambda p, b: (p, 0)),
        pl.BlockSpec((types, H), lambda p, b: (0, 0)),
        pl.BlockSpec((1, H), lambda p, b: (0, 0)),
        pl.BlockSpec((1, H), lambda p, b: (0, 0)),
    ]
    if with_prev:
        in_specs.append(pl.BlockSpec(memory_space=pl.ANY))
    return pl.pallas_call(
        _ln_body,
        grid=(pos_blocks, stage_batch),
        in_specs=in_specs,
        out_specs=pl.BlockSpec(
            (TBLK, H), lambda p, b: (blk0 + b * pos_blocks + p, 0)),
        out_shape=jax.ShapeDtypeStruct((n_tokens, H), jnp.float32),
        input_output_aliases={6: 0} if with_prev else {},
    )


def kernel(token_ids, token_type_ids, word_emb, pos_emb, type_emb, gamma, beta):
    batch, seq = token_ids.shape
    vocab, hidden = word_emb.shape
    types = type_emb.shape[0]
    n_tokens = batch * seq

    tok = token_ids.astype(jnp.int32)
    tt3 = token_type_ids.reshape(n_tokens // TBLK, 1, TBLK).astype(jnp.int32)
    gam = gamma.reshape(1, hidden)
    bet = beta.reshape(1, hidden)

    stage_tok = [b * seq for b in STAGE_BATCHES]
    assert sum(stage_tok) == n_tokens
    bases = [sum(stage_tok[:h]) for h in range(len(stage_tok))]

    gathered = [
        _make_sc_gather(n_tokens, stage_tok[h], bases[h])(tok, word_emb)
        for h in range(len(stage_tok))]

    out = None
    for h in range(len(stage_tok)):
        args = [gathered[h], tt3, pos_emb, type_emb, gam, bet]
        if out is not None:
            args.append(out)
        out = _make_tc_ln(n_tokens, stage_tok[h], seq, types,
                          bases[h] // TBLK, out is not None)(*args)
    return out.reshape(batch, seq, hidden)
